# baseline (speedup 1.0000x reference)
import functools

import jax
import jax.numpy as jnp
from jax import lax
from jax.experimental import pallas as pl
from jax.experimental.pallas import tpu as pltpu
from jax.experimental.pallas import tpu_sc as plsc

B = 16384
D = 64
NC = 2
NS = 16
NW = NC * NS
BPW = B // NW


def _sc_gather_body(user_hbm, pos_hbm, neg_hbm, ut_hbm, it_hbm,
                    u_out, p_out, n_out,
                    idx_v, rows_v, sem):
    wid = lax.axis_index("s") * NC + lax.axis_index("c")
    base = wid * BPW
    pltpu.sync_copy(user_hbm.at[pl.ds(base, BPW)], idx_v)
    pltpu.async_copy(ut_hbm.at[idx_v], rows_v, sem).wait()
    pltpu.sync_copy(rows_v, u_out.at[pl.ds(base, BPW)])
    pltpu.sync_copy(pos_hbm.at[pl.ds(base, BPW)], idx_v)
    pltpu.async_copy(it_hbm.at[idx_v], rows_v, sem).wait()
    pltpu.sync_copy(rows_v, p_out.at[pl.ds(base, BPW)])
    pltpu.sync_copy(neg_hbm.at[pl.ds(base, BPW)], idx_v)
    pltpu.async_copy(it_hbm.at[idx_v], rows_v, sem).wait()
    pltpu.sync_copy(rows_v, n_out.at[pl.ds(base, BPW)])


_sc_gather = pl.kernel(
    _sc_gather_body,
    out_type=(
        jax.ShapeDtypeStruct((B, D), jnp.float32),
        jax.ShapeDtypeStruct((B, D), jnp.float32),
        jax.ShapeDtypeStruct((B, D), jnp.float32),
    ),
    mesh=plsc.VectorSubcoreMesh(core_axis_name="c", subcore_axis_name="s"),
    scratch_types=[
        pltpu.VMEM((BPW,), jnp.int32),
        pltpu.VMEM((BPW, D), jnp.float32),
        pltpu.SemaphoreType.DMA,
    ],
    compiler_params=pltpu.CompilerParams(use_tc_tiling_on_sc=False),
)


_TC_BLK = 2048


def _tc_loss_body(u_ref, p_ref, n_ref, out_ref):
    @pl.when(pl.program_id(0) == 0)
    def _():
        out_ref[0, 0] = 0.0

    u = u_ref[...]
    t = jnp.sum(u * (p_ref[...] - n_ref[...]), axis=1)
    loss = jnp.maximum(-t, 0.0) + jnp.log1p(jnp.exp(-jnp.abs(t)))
    out_ref[0, 0] += jnp.sum(loss)


_tc_loss = pl.pallas_call(
    _tc_loss_body,
    grid=(B // _TC_BLK,),
    in_specs=[
        pl.BlockSpec((_TC_BLK, D), lambda i: (i, 0)),
        pl.BlockSpec((_TC_BLK, D), lambda i: (i, 0)),
        pl.BlockSpec((_TC_BLK, D), lambda i: (i, 0)),
    ],
    out_specs=pl.BlockSpec(memory_space=pltpu.SMEM),
    out_shape=jax.ShapeDtypeStruct((1, 1), jnp.float32),
)


def kernel(user, pos, neg, user_table, item_table):
    u_rows, p_rows, n_rows = _sc_gather(user, pos, neg, user_table, item_table)
    loss = _tc_loss(u_rows, p_rows, n_rows)
    return loss[0, 0]


# speedup vs baseline: 1.5891x; 1.5891x over previous
import functools

import jax
import jax.numpy as jnp
from jax import lax
from jax.experimental import pallas as pl
from jax.experimental.pallas import tpu as pltpu
from jax.experimental.pallas import tpu_sc as plsc

B = 16384
D = 64
NC = 2
NS = 16
NW = NC * NS
BPW = B // NW


def _sc_gather_body(user_hbm, pos_hbm, neg_hbm, ut_hbm, it_hbm,
                    u_out, p_out, n_out,
                    idx_v, rows_v, sem):
    wid = lax.axis_index("s") * NC + lax.axis_index("c")
    base = wid * BPW

    def one_table(idx_hbm, table_hbm, out_hbm):
        pltpu.sync_copy(idx_hbm.at[pl.ds(base, BPW)], idx_v)

        def fire(g, carry):
            v = idx_v[pl.ds(g * 16, 16)]
            for j in range(16):
                pltpu.make_async_copy(
                    table_hbm.at[v[j]], rows_v.at[g * 16 + j], sem).start()
            return carry

        lax.fori_loop(0, BPW // 16, fire, 0)
        pltpu.make_async_copy(table_hbm.at[pl.ds(0, BPW)], rows_v, sem).wait()
        pltpu.sync_copy(rows_v, out_hbm.at[pl.ds(base, BPW)])

    one_table(user_hbm, ut_hbm, u_out)
    one_table(pos_hbm, it_hbm, p_out)
    one_table(neg_hbm, it_hbm, n_out)


_sc_gather = pl.kernel(
    _sc_gather_body,
    out_type=(
        jax.ShapeDtypeStruct((B, D), jnp.float32),
        jax.ShapeDtypeStruct((B, D), jnp.float32),
        jax.ShapeDtypeStruct((B, D), jnp.float32),
    ),
    mesh=plsc.VectorSubcoreMesh(core_axis_name="c", subcore_axis_name="s"),
    scratch_types=[
        pltpu.VMEM((BPW,), jnp.int32),
        pltpu.VMEM((BPW, D), jnp.float32),
        pltpu.SemaphoreType.DMA,
    ],
)


_TC_BLK = 2048


def _tc_loss_body(u_ref, p_ref, n_ref, out_ref):
    @pl.when(pl.program_id(0) == 0)
    def _():
        out_ref[0, 0] = 0.0

    u = u_ref[...]
    t = jnp.sum(u * (p_ref[...] - n_ref[...]), axis=1)
    loss = jnp.maximum(-t, 0.0) + jnp.log1p(jnp.exp(-jnp.abs(t)))
    out_ref[0, 0] += jnp.sum(loss)


_tc_loss = pl.pallas_call(
    _tc_loss_body,
    grid=(B // _TC_BLK,),
    in_specs=[
        pl.BlockSpec((_TC_BLK, D), lambda i: (i, 0)),
        pl.BlockSpec((_TC_BLK, D), lambda i: (i, 0)),
        pl.BlockSpec((_TC_BLK, D), lambda i: (i, 0)),
    ],
    out_specs=pl.BlockSpec(memory_space=pltpu.SMEM),
    out_shape=jax.ShapeDtypeStruct((1, 1), jnp.float32),
)


def kernel(user, pos, neg, user_table, item_table):
    u_rows, p_rows, n_rows = _sc_gather(user, pos, neg, user_table, item_table)
    loss = _tc_loss(u_rows, p_rows, n_rows)
    return loss[0, 0]


# speedup vs baseline: 1.5899x; 1.0005x over previous
import jax
import jax.numpy as jnp
from jax import lax
from jax.experimental import pallas as pl
from jax.experimental.pallas import tpu as pltpu
from jax.experimental.pallas import tpu_sc as plsc

B = 16384
D = 64
NC = 2
NS = 16
NW = NC * NS
BPW = B // NW


def _sc_gather_body(user_hbm, pos_hbm, neg_hbm, ut_hbm, it_hbm,
                    u_out, p_out, n_out,
                    idx_v, rows_v, sem):
    wid = lax.axis_index("s") * NC + lax.axis_index("c")
    base = wid * BPW

    def one_table(idx_hbm, table_hbm, out_hbm):
        pltpu.sync_copy(idx_hbm.at[pl.ds(base, BPW)], idx_v)

        def fire(g, carry):
            v = idx_v[pl.ds(g * 16, 16)]
            for j in range(16):
                pltpu.make_async_copy(
                    table_hbm.at[v[j]], rows_v.at[g * 16 + j], sem).start()
            return carry

        lax.fori_loop(0, BPW // 16, fire, 0)
        pltpu.make_async_copy(table_hbm.at[pl.ds(0, BPW)], rows_v, sem).wait()
        pltpu.sync_copy(rows_v, out_hbm.at[pl.ds(base, BPW)])

    one_table(user_hbm, ut_hbm, u_out)
    one_table(pos_hbm, it_hbm, p_out)
    one_table(neg_hbm, it_hbm, n_out)


_sc_gather = pl.kernel(
    _sc_gather_body,
    out_type=(
        jax.ShapeDtypeStruct((B, D), jnp.float32),
        jax.ShapeDtypeStruct((B, D), jnp.float32),
        jax.ShapeDtypeStruct((B, D), jnp.float32),
    ),
    mesh=plsc.VectorSubcoreMesh(core_axis_name="c", subcore_axis_name="s"),
    scratch_types=[
        pltpu.VMEM((BPW,), jnp.int32),
        pltpu.VMEM((BPW, D), jnp.float32),
        pltpu.SemaphoreType.DMA,
    ],
)


_TC_BLK = 2048


def _tc_loss_body(u_ref, p_ref, n_ref, out_ref):
    @pl.when(pl.program_id(0) == 0)
    def _():
        out_ref[0, 0] = 0.0

    u = u_ref[...]
    t = jnp.sum(u * (p_ref[...] - n_ref[...]), axis=1)
    loss = jnp.maximum(-t, 0.0) + jnp.log1p(jnp.exp(-jnp.abs(t)))
    out_ref[0, 0] += jnp.sum(loss)


_tc_loss = pl.pallas_call(
    _tc_loss_body,
    grid=(B // _TC_BLK,),
    in_specs=[
        pl.BlockSpec((_TC_BLK, D), lambda i: (i, 0)),
        pl.BlockSpec((_TC_BLK, D), lambda i: (i, 0)),
        pl.BlockSpec((_TC_BLK, D), lambda i: (i, 0)),
    ],
    out_specs=pl.BlockSpec(memory_space=pltpu.SMEM),
    out_shape=jax.ShapeDtypeStruct((1, 1), jnp.float32),
)


def kernel(user, pos, neg, user_table, item_table):
    u_rows, p_rows, n_rows = _sc_gather(user, pos, neg, user_table, item_table)
    loss = _tc_loss(u_rows, p_rows, n_rows)
    return loss[0, 0]
